# baseline (speedup 1.0000x reference)
import functools

import jax
import jax.numpy as jnp
from jax import lax
from jax.experimental import pallas as pl
from jax.experimental.pallas import tpu as pltpu
from jax.experimental.pallas import tpu_sc as plsc

_LANES = 16
_NBUF = 8
_G = 5


@functools.partial(jax.jit, static_argnums=(3, 4, 5))
def _embed_call(ids_flat, token_embedding, position_embedding, B, S, D):
    NC, NS = 2, 16
    NW = NC * NS
    R = B * S
    RPW = R // NW
    C = 8
    NCH = RPW // C
    NR = NCH // _NBUF
    mesh = plsc.VectorSubcoreMesh(core_axis_name="c", subcore_axis_name="s")

    @functools.partial(
        pl.kernel,
        mesh=mesh,
        out_type=jax.ShapeDtypeStruct((R, D), jnp.float32),
        scratch_types=[
            pltpu.VMEM((RPW,), jnp.int32),
            pltpu.VMEM((S, D), jnp.float32),
        ]
        + [pltpu.VMEM((C, D), jnp.float32)] * _NBUF
        + [pltpu.SemaphoreType.DMA] * (2 * _NBUF),
    )
    def k(ids_hbm, tok_hbm, pos_hbm, out_hbm, idx_v, pos_v, *bs):
        bufs = bs[:_NBUF]
        gs = bs[_NBUF:2 * _NBUF]
        os_ = bs[2 * _NBUF:]
        wid = lax.axis_index("s") * NC + lax.axis_index("c")
        base = wid * RPW
        pltpu.sync_copy(ids_hbm.at[pl.ds(base, RPW)], idx_v)
        pltpu.sync_copy(pos_hbm, pos_v)

        def gather_start(kk, b):
            pltpu.async_copy(tok_hbm.at[idx_v.at[pl.ds(kk * C, C)]], bufs[b], gs[b])

        def gather_wait(kk, b):
            pltpu.make_async_copy(
                tok_hbm.at[idx_v.at[pl.ds(kk * C, C)]], bufs[b], gs[b]
            ).wait()

        def out_start(kk, b):
            pltpu.async_copy(bufs[b], out_hbm.at[pl.ds(base + kk * C, C)], os_[b])

        def out_wait(b):
            pltpu.make_async_copy(
                bufs[b], out_hbm.at[pl.ds(base, C)], os_[b]
            ).wait()

        def add_rows(kk, b):
            buf = bufs[b]
            p0 = lax.rem(kk * C, S)

            @plsc.parallel_loop(0, C, unroll=2)
            def _(i):
                p = p0 + i
                p = jnp.where(p >= S, p - S, p)
                for l in range(D // _LANES):
                    sl = pl.ds(l * _LANES, _LANES)
                    buf[i, sl] = buf[i, sl] + pos_v[p, sl]

        def chunk_step(kk, b):
            bg = (b + _G) % _NBUF

            @pl.when(kk >= _NBUF - _G)
            def _():
                out_wait(bg)

            @pl.when(kk < NCH - _G)
            def _():
                gather_start(kk + _G, bg)

            gather_wait(kk, b)
            add_rows(kk, b)
            out_start(kk, b)

        for j in range(_G):
            gather_start(j, j)

        def round_body(t, _):
            for b in range(_NBUF):
                chunk_step(t * _NBUF + b, b)
            return 0

        lax.fori_loop(0, NR, round_body, 0)

        for b in range(_G, _NBUF):
            out_wait(b)

    return k(ids_flat, token_embedding, position_embedding)


def kernel(input_ids, token_embedding, position_embedding):
    B, S = input_ids.shape
    _, D = token_embedding.shape
    out = _embed_call(
        input_ids.reshape(-1), token_embedding, position_embedding, B, S, D
    )
    return out.reshape(B, S, D)


# speedup vs baseline: 1.2476x; 1.2476x over previous
import functools

import jax
import jax.numpy as jnp
from jax import lax
from jax.experimental import pallas as pl
from jax.experimental.pallas import tpu as pltpu
from jax.experimental.pallas import tpu_sc as plsc

_LANES = 16
_NBUF = 4
_G = 2
_SPLIT = 2


@functools.partial(jax.jit, static_argnums=(3, 4, 5))
def _embed_call(ids_flat, token_embedding, position_embedding, B, S, D):
    NC, NS = 2, 16
    NW = NC * NS
    R = B * S
    RPW = R // NW
    C = 16
    NCH = RPW // C
    NR = NCH // _NBUF
    mesh = plsc.VectorSubcoreMesh(core_axis_name="c", subcore_axis_name="s")

    @functools.partial(
        pl.kernel,
        mesh=mesh,
        out_type=jax.ShapeDtypeStruct((R, D), jnp.float32),
        scratch_types=[
            pltpu.VMEM((RPW,), jnp.int32),
            pltpu.VMEM((S, D), jnp.float32),
        ]
        + [pltpu.VMEM((C, D), jnp.float32)] * _NBUF
        + [pltpu.SemaphoreType.DMA] * ((_SPLIT + 1) * _NBUF),
    )
    def k(ids_hbm, tok_hbm, pos_hbm, out_hbm, idx_v, pos_v, *bs):
        bufs = bs[:_NBUF]
        gs = bs[_NBUF:(_SPLIT + 1) * _NBUF]
        os_ = bs[(_SPLIT + 1) * _NBUF:]
        wid = lax.axis_index("s") * NC + lax.axis_index("c")
        base = wid * RPW
        pltpu.sync_copy(ids_hbm.at[pl.ds(base, RPW)], idx_v)
        pltpu.sync_copy(pos_hbm, pos_v)

        H = C // _SPLIT

        def gather_start(kk, b):
            for h in range(_SPLIT):
                pltpu.async_copy(
                    tok_hbm.at[idx_v.at[pl.ds(kk * C + h * H, H)]],
                    bufs[b].at[pl.ds(h * H, H)],
                    gs[b * _SPLIT + h],
                )

        def gather_wait(kk, b):
            for h in range(_SPLIT):
                pltpu.make_async_copy(
                    tok_hbm.at[idx_v.at[pl.ds(kk * C + h * H, H)]],
                    bufs[b].at[pl.ds(h * H, H)],
                    gs[b * _SPLIT + h],
                ).wait()

        def out_start(kk, b):
            pltpu.async_copy(bufs[b], out_hbm.at[pl.ds(base + kk * C, C)], os_[b])

        def out_wait(b):
            pltpu.make_async_copy(
                bufs[b], out_hbm.at[pl.ds(base, C)], os_[b]
            ).wait()

        def add_rows(kk, b):
            buf = bufs[b]
            p0 = lax.rem(kk * C, S)

            @plsc.parallel_loop(0, C, unroll=2)
            def _(i):
                p = p0 + i
                p = jnp.where(p >= S, p - S, p)
                for l in range(D // _LANES):
                    sl = pl.ds(l * _LANES, _LANES)
                    buf[i, sl] = buf[i, sl] + pos_v[p, sl]

        def chunk_step(kk, b):
            bg = (b + _G) % _NBUF

            @pl.when(kk >= _NBUF - _G)
            def _():
                out_wait(bg)

            @pl.when(kk < NCH - _G)
            def _():
                gather_start(kk + _G, bg)

            gather_wait(kk, b)
            add_rows(kk, b)
            out_start(kk, b)

        for j in range(_G):
            gather_start(j, j)

        def round_body(t, _):
            for b in range(_NBUF):
                chunk_step(t * _NBUF + b, b)
            return 0

        lax.fori_loop(0, NR, round_body, 0)

        for b in range(_G, _NBUF):
            out_wait(b)

    return k(ids_flat, token_embedding, position_embedding)


def kernel(input_ids, token_embedding, position_embedding):
    B, S = input_ids.shape
    _, D = token_embedding.shape
    out = _embed_call(
        input_ids.reshape(-1), token_embedding, position_embedding, B, S, D
    )
    return out.reshape(B, S, D)
